# baseline (speedup 1.0000x reference)
import jax
import jax.numpy as jnp
from jax import lax
from jax.experimental import pallas as pl
from jax.experimental.pallas import tpu as pltpu

T = 2048
C = 768
H = 1024
NE = 64
K = 2
NR = T * K
TM = 128
G = NR // TM + NE
NH = 2
H2 = H // NH


def _router_body(x_ref, wg_ref, w_ref, i_ref):
    logits = jnp.dot(x_ref[...], wg_ref[...],
                     preferred_element_type=jnp.float32)
    col = lax.broadcasted_iota(jnp.int32, (T, NE), 1)
    m1 = jnp.max(logits, axis=1, keepdims=True)
    a1 = jnp.min(jnp.where(logits == m1, col, NE), axis=1, keepdims=True)
    l2 = jnp.where(col == a1, -1e30, logits)
    m2 = jnp.max(l2, axis=1, keepdims=True)
    a2 = jnp.min(jnp.where(l2 == m2, col, NE), axis=1, keepdims=True)
    z = jnp.exp(m2 - m1)
    w1 = 1.0 / (1.0 + z)
    w_ref[...] = jnp.concatenate([w1, z * w1], axis=1)
    i_ref[...] = jnp.concatenate([a1, a2], axis=1)


def _ffn_body(em, tk, ws, rw, x_ref, w1_ref, w2_ref, w3_ref, out_ref,
              xs_ref, xs2_ref, y_ref):
    g = pl.program_id(0)
    n = pl.program_id(1)

    @pl.when((g == 0) & (n == 0))
    def _():
        out_ref[...] = jnp.zeros_like(out_ref)

    rows = rw[g]
    base = g * TM

    @pl.when(n == 0)
    def _():
        def gather(i, carry):
            t = tk[base + i]
            w = ws[base + i]
            row = x_ref[pl.ds(t, 1), :]
            xs_ref[pl.ds(i, 1), :] = row
            xs2_ref[pl.ds(i, 1), :] = w * row
            return carry

        lax.fori_loop(0, rows, gather, 0)

    @pl.when(rows > 0)
    def _():
        a = jnp.dot(xs_ref[...], w1_ref[0], preferred_element_type=jnp.float32)
        b = jnp.dot(xs2_ref[...], w2_ref[0], preferred_element_type=jnp.float32)
        h = a * (1.0 / (1.0 + jnp.exp(-a))) * b
        part = jnp.dot(h, w3_ref[0], preferred_element_type=jnp.float32)

        @pl.when(n == 0)
        def _():
            y_ref[...] = part

        @pl.when(n > 0)
        def _():
            y_ref[...] = y_ref[...] + part

        @pl.when(n == NH - 1)
        def _():
            def scat(i, carry):
                t = tk[base + i]
                out_ref[pl.ds(t, 1), :] = (out_ref[pl.ds(t, 1), :]
                                           + y_ref[pl.ds(i, 1), :])
                return carry

            lax.fori_loop(0, rows, scat, 0)


def kernel(x, Wg, W1, W2, W3):
    xf = x.reshape(T, C)

    wts, idx = pl.pallas_call(
        _router_body,
        out_shape=(jax.ShapeDtypeStruct((T, K), jnp.float32),
                   jax.ShapeDtypeStruct((T, K), jnp.int32)),
    )(xf, Wg)

    e_flat = idx.reshape(-1)
    order = jnp.argsort(e_flat).astype(jnp.int32)
    counts = jnp.zeros((NE,), jnp.int32).at[e_flat].add(1)
    starts = jnp.concatenate(
        [jnp.zeros((1,), jnp.int32), jnp.cumsum(counts)[:-1].astype(jnp.int32)])
    ntiles = (counts + TM - 1) // TM
    tend = jnp.cumsum(ntiles).astype(jnp.int32)
    total = tend[-1]
    gs = jnp.arange(G, dtype=jnp.int32)
    gc = jnp.minimum(gs, total - 1)
    e_act = jnp.searchsorted(tend, gc, side='right').astype(jnp.int32)
    within = gc - (tend[e_act] - ntiles[e_act])
    src_start = starts[e_act] + within * TM
    rows = jnp.where(gs < total,
                     jnp.minimum(TM, counts[e_act] - within * TM),
                     0).astype(jnp.int32)
    pos = src_start[:, None] + jnp.arange(TM, dtype=jnp.int32)[None, :]
    valid = jnp.arange(TM, dtype=jnp.int32)[None, :] < rows[:, None]
    f = order[jnp.clip(pos, 0, NR - 1)]
    tok_pad = jnp.where(valid, f // K, 0).reshape(-1).astype(jnp.int32)
    ws_pad = jnp.where(valid, wts.reshape(-1)[f], 0.0).reshape(-1)

    grid_spec = pltpu.PrefetchScalarGridSpec(
        num_scalar_prefetch=4,
        grid=(G, NH),
        in_specs=[
            pl.BlockSpec((T, C), lambda g, n, em, tk, ws, rw: (0, 0)),
            pl.BlockSpec((1, C, H2), lambda g, n, em, tk, ws, rw: (em[g], 0, n)),
            pl.BlockSpec((1, C, H2), lambda g, n, em, tk, ws, rw: (em[g], 0, n)),
            pl.BlockSpec((1, H2, C), lambda g, n, em, tk, ws, rw: (em[g], n, 0)),
        ],
        out_specs=pl.BlockSpec((T, C), lambda g, n, em, tk, ws, rw: (0, 0)),
        scratch_shapes=[
            pltpu.VMEM((TM, C), jnp.float32),
            pltpu.VMEM((TM, C), jnp.float32),
            pltpu.VMEM((TM, C), jnp.float32),
        ],
    )

    out = pl.pallas_call(
        _ffn_body,
        grid_spec=grid_spec,
        out_shape=jax.ShapeDtypeStruct((T, C), jnp.float32),
        compiler_params=pltpu.CompilerParams(
            dimension_semantics=("arbitrary", "arbitrary")),
    )(e_act, tok_pad, ws_pad, rows, xf, W1, W2, W3)

    return out.reshape(1, T, C)


# speedup vs baseline: 1.3540x; 1.3540x over previous
import jax
import jax.numpy as jnp
from jax import lax
from jax.experimental import pallas as pl
from jax.experimental.pallas import tpu as pltpu

T = 2048
C = 768
H = 1024
NE = 64
K = 2
NR = T * K
TM = 128
G = NR // TM + NE


def _router_body(x_ref, wg_ref, w_ref, i_ref):
    logits = jnp.dot(x_ref[...], wg_ref[...],
                     preferred_element_type=jnp.float32)
    col = lax.broadcasted_iota(jnp.int32, (T, NE), 1)
    m1 = jnp.max(logits, axis=1, keepdims=True)
    a1 = jnp.min(jnp.where(logits == m1, col, NE), axis=1, keepdims=True)
    l2 = jnp.where(col == a1, -1e30, logits)
    m2 = jnp.max(l2, axis=1, keepdims=True)
    a2 = jnp.min(jnp.where(l2 == m2, col, NE), axis=1, keepdims=True)
    z = jnp.exp(m2 - m1)
    w1 = 1.0 / (1.0 + z)
    w_ref[...] = jnp.concatenate([w1, z * w1], axis=1)
    i_ref[...] = jnp.concatenate([a1, a2], axis=1)


def _start_fetch(e, s, w1_any, w2_any, w3_any, w1b, w2b, w3b, sem):
    pltpu.make_async_copy(w1_any.at[e], w1b.at[s], sem.at[0, s]).start()
    pltpu.make_async_copy(w2_any.at[e], w2b.at[s], sem.at[1, s]).start()
    pltpu.make_async_copy(w3_any.at[e], w3b.at[s], sem.at[2, s]).start()


def _wait_fetch(e, s, w1_any, w2_any, w3_any, w1b, w2b, w3b, sem):
    pltpu.make_async_copy(w1_any.at[e], w1b.at[s], sem.at[0, s]).wait()
    pltpu.make_async_copy(w2_any.at[e], w2b.at[s], sem.at[1, s]).wait()
    pltpu.make_async_copy(w3_any.at[e], w3b.at[s], sem.at[2, s]).wait()


def _ffn_body(em, tk, ws, rw, uslot, fneed,
              x_ref, w1_any, w2_any, w3_any, out_ref,
              w1b, w2b, w3b, xs_ref, xs2_ref, y_ref, sem):
    g = pl.program_id(0)
    s = uslot[g]

    @pl.when(g == 0)
    def _():
        out_ref[...] = jnp.zeros_like(out_ref)
        _start_fetch(em[0], 0, w1_any, w2_any, w3_any, w1b, w2b, w3b, sem)

    gn = jnp.minimum(g + 1, G - 1)

    @pl.when((g + 1 < G) & (fneed[gn] == 1))
    def _():
        _start_fetch(em[gn], uslot[gn],
                     w1_any, w2_any, w3_any, w1b, w2b, w3b, sem)

    rows = rw[g]
    base = g * TM

    def gather(i, carry):
        t = tk[base + i]
        w = ws[base + i]
        row = x_ref[pl.ds(t, 1), :]
        xs_ref[pl.ds(i, 1), :] = row
        xs2_ref[pl.ds(i, 1), :] = w * row
        return carry

    lax.fori_loop(0, rows, gather, 0)

    @pl.when(fneed[g] == 1)
    def _():
        _wait_fetch(em[g], s, w1_any, w2_any, w3_any, w1b, w2b, w3b, sem)

    @pl.when(rows > 0)
    def _():
        xb = xs_ref[...].astype(jnp.bfloat16)
        x2b = xs2_ref[...].astype(jnp.bfloat16)
        a = jnp.dot(xb, w1b[s].astype(jnp.bfloat16),
                    preferred_element_type=jnp.float32)
        b = jnp.dot(x2b, w2b[s].astype(jnp.bfloat16),
                    preferred_element_type=jnp.float32)
        h = a * (1.0 / (1.0 + jnp.exp(-a))) * b
        y_ref[...] = jnp.dot(h.astype(jnp.bfloat16),
                             w3b[s].astype(jnp.bfloat16),
                             preferred_element_type=jnp.float32)

        def scat(i, carry):
            t = tk[base + i]
            out_ref[pl.ds(t, 1), :] = (out_ref[pl.ds(t, 1), :]
                                       + y_ref[pl.ds(i, 1), :])
            return carry

        lax.fori_loop(0, rows, scat, 0)


def kernel(x, Wg, W1, W2, W3):
    xf = x.reshape(T, C)

    wts, idx = pl.pallas_call(
        _router_body,
        out_shape=(jax.ShapeDtypeStruct((T, K), jnp.float32),
                   jax.ShapeDtypeStruct((T, K), jnp.int32)),
    )(xf, Wg)

    e_flat = idx.reshape(-1)
    order = jnp.argsort(e_flat).astype(jnp.int32)
    counts = jnp.zeros((NE,), jnp.int32).at[e_flat].add(1)
    starts = jnp.concatenate(
        [jnp.zeros((1,), jnp.int32), jnp.cumsum(counts)[:-1].astype(jnp.int32)])
    ntiles = (counts + TM - 1) // TM
    tend = jnp.cumsum(ntiles).astype(jnp.int32)
    total = tend[-1]
    gs = jnp.arange(G, dtype=jnp.int32)
    gc = jnp.minimum(gs, total - 1)
    e_act = jnp.searchsorted(tend, gc, side='right').astype(jnp.int32)
    within = gc - (tend[e_act] - ntiles[e_act])
    src_start = starts[e_act] + within * TM
    rows = jnp.where(gs < total,
                     jnp.minimum(TM, counts[e_act] - within * TM),
                     0).astype(jnp.int32)
    pos = src_start[:, None] + jnp.arange(TM, dtype=jnp.int32)[None, :]
    valid = jnp.arange(TM, dtype=jnp.int32)[None, :] < rows[:, None]
    f = order[jnp.clip(pos, 0, NR - 1)]
    tok_pad = jnp.where(valid, f // K, 0).reshape(-1).astype(jnp.int32)
    ws_pad = jnp.where(valid, wts.reshape(-1)[f], 0.0).reshape(-1)

    fneed = jnp.concatenate(
        [jnp.ones((1,), jnp.int32),
         (e_act[1:] != e_act[:-1]).astype(jnp.int32)])
    uslot = (jnp.cumsum(fneed).astype(jnp.int32) - 1) % 2

    grid_spec = pltpu.PrefetchScalarGridSpec(
        num_scalar_prefetch=6,
        grid=(G,),
        in_specs=[
            pl.BlockSpec((T, C), lambda g, *p: (0, 0)),
            pl.BlockSpec(memory_space=pl.ANY),
            pl.BlockSpec(memory_space=pl.ANY),
            pl.BlockSpec(memory_space=pl.ANY),
        ],
        out_specs=pl.BlockSpec((T, C), lambda g, *p: (0, 0)),
        scratch_shapes=[
            pltpu.VMEM((2, C, H), jnp.float32),
            pltpu.VMEM((2, C, H), jnp.float32),
            pltpu.VMEM((2, H, C), jnp.float32),
            pltpu.VMEM((TM, C), jnp.float32),
            pltpu.VMEM((TM, C), jnp.float32),
            pltpu.VMEM((TM, C), jnp.float32),
            pltpu.SemaphoreType.DMA((3, 2)),
        ],
    )

    out = pl.pallas_call(
        _ffn_body,
        grid_spec=grid_spec,
        out_shape=jax.ShapeDtypeStruct((T, C), jnp.float32),
        compiler_params=pltpu.CompilerParams(
            dimension_semantics=("arbitrary",)),
    )(e_act, tok_pad, ws_pad, rows, uslot, fneed, xf, W1, W2, W3)

    return out.reshape(1, T, C)
